# baseline (speedup 1.0000x reference)
import functools

import jax
import jax.numpy as jnp
from jax import lax
from jax.experimental import pallas as pl
from jax.experimental.pallas import tpu as pltpu
from jax.experimental.pallas import tpu_sc as plsc

N = 10000
NP = 10240
E = 320000
D = 128
G = 64
NC, NS, L = 2, 16, 16
NT = NC * NS
CH = 128
CPT = 80
EPAD = NT * CPT * CH
NB = 2
NGR = CPT // NB
RPT = NP // NS

_mesh = plsc.VectorSubcoreMesh(core_axis_name="c", subcore_axis_name="s")


def _newton_rsqrt(d):
    i = plsc.bitcast(d, jnp.int32)
    i = jnp.int32(0x5F3759DF) - lax.shift_right_logical(i, 1)
    y = plsc.bitcast(i, jnp.float32)
    for _ in range(4):
        y = y * (1.5 - 0.5 * d * y * y)
    return y


@functools.partial(
    pl.kernel,
    out_type=jax.ShapeDtypeStruct((NP,), jnp.float32),
    mesh=_mesh,
    compiler_params=pltpu.CompilerParams(needs_layout_passes=False),
    scratch_types=[
        pltpu.VMEM((NP,), jnp.float32),
        pltpu.VMEM((CPT, CH), jnp.int32),
        pltpu.VMEM((RPT,), jnp.float32),
        pltpu.VMEM((RPT,), jnp.float32),
        pltpu.VMEM_SHARED((NS, NP), jnp.float32),
    ],
)
def _sc_deg(dstp_ref, dis_ref, hist, dstbuf, acc, tmp, hsh):
    c = lax.axis_index("c")
    s = lax.axis_index("s")
    ones = jnp.ones((L,), jnp.float32)

    @pl.when(c == 0)
    def _():
        def zhist(i, _):
            hist[pl.ds(i * L, L)] = jnp.zeros((L,), jnp.float32)
            return _
        lax.fori_loop(0, NP // L, zhist, 0)

        for t in range(2):
            w = s + t * NS
            pltpu.sync_copy(dstp_ref.at[w], dstbuf)

            def brow(r, _):
                for jj in range(CH // L):
                    idx = dstbuf[r, pl.ds(jj * L, L)]
                    plsc.addupdate_scatter(hist, [idx], ones)
                return _
            lax.fori_loop(0, CPT, brow, 0)

        pltpu.sync_copy(hist, hsh.at[s])
        plsc.subcore_barrier()

        base = s * RPT
        def zacc(i, _):
            acc[pl.ds(i * L, L)] = jnp.zeros((L,), jnp.float32)
            return _
        lax.fori_loop(0, RPT // L, zacc, 0)
        for k in range(NS):
            pltpu.sync_copy(hsh.at[k, pl.ds(base, RPT)], tmp)

            def radd(i, _):
                sl = pl.ds(i * L, L)
                acc[sl] = acc[sl] + tmp[sl]
                return _
            lax.fori_loop(0, RPT // L, radd, 0)

        def wdis(i, _):
            sl = pl.ds(i * L, L)
            deg = acc[sl] + 1.0
            y = _newton_rsqrt(deg)
            ids = base + i * L + lax.iota(jnp.int32, L)
            tmp[sl] = jnp.where(ids < N, y, 0.0)
            return _
        lax.fori_loop(0, RPT // L, wdis, 0)
        pltpu.sync_copy(tmp, dis_ref.at[pl.ds(base, RPT)])


@functools.partial(
    pl.kernel,
    out_type=jax.ShapeDtypeStruct((NC, NP, D), jnp.float32),
    mesh=_mesh,
    scratch_types=(
        [pltpu.VMEM((CH, D), jnp.float32) for _ in range(NB)]
        + [pltpu.VMEM((CH,), jnp.int32) for _ in range(NB)]
        + [pltpu.VMEM((CH,), jnp.int32) for _ in range(NB)]
        + [pltpu.SemaphoreType.DMA for _ in range(NB)]
        + [pltpu.VMEM_SHARED((NP, D), jnp.float32)]
    ),
)
def _sc_agg(srcp_ref, dstp_ref, hs_ref, zer_ref, z_ref, *scr):
    rows = scr[0:NB]
    isrc = scr[NB:2 * NB]
    idst = scr[2 * NB:3 * NB]
    gsem = scr[3 * NB:4 * NB]
    z_sh = scr[4 * NB]

    c = lax.axis_index("c")
    s = lax.axis_index("s")
    w = c * NS + s
    base = s * RPT

    for k in range(RPT // CH):
        pltpu.sync_copy(zer_ref.at[pl.ds(base + k * CH, CH)], rows[0])
        pltpu.sync_copy(rows[0], z_sh.at[pl.ds(base + k * CH, CH)])

    plsc.subcore_barrier()

    for b in range(NB):
        pltpu.sync_copy(srcp_ref.at[w, b], isrc[b])
        pltpu.sync_copy(dstp_ref.at[w, b], idst[b])
        pltpu.async_copy(hs_ref.at[isrc[b]], rows[b], gsem[b])

    def group(g, carry):
        for b in range(NB):
            pltpu.make_async_copy(hs_ref.at[isrc[b]], rows[b], gsem[b]).wait()
            pltpu.sync_copy(rows[b], z_sh.at[idst[b]], add=True)

            @pl.when(g < NGR - 1)
            def _(b=b):
                jn = (g + 1) * NB + b
                pltpu.sync_copy(srcp_ref.at[w, jn], isrc[b])
                pltpu.sync_copy(dstp_ref.at[w, jn], idst[b])
                pltpu.async_copy(hs_ref.at[isrc[b]], rows[b], gsem[b])
        return carry

    lax.fori_loop(0, NGR, group, 0)
    plsc.subcore_barrier()

    for k in range(RPT // CH):
        sl = pl.ds(base + k * CH, CH)
        pltpu.sync_copy(z_sh.at[sl], z_ref.at[c, sl])


def _tc_first_body(x_ref, w_ref, dis_ref, hs_ref):
    h = lax.dot_general(x_ref[...], w_ref[...], (((1,), (0,)), ((), ())),
                        preferred_element_type=jnp.float32,
                        precision=lax.Precision.HIGHEST)
    hs_ref[...] = h * dis_ref[...]


def _tc_mid_body(z_ref, hsin_ref, dis_ref, w_ref, b_ref, hs_ref):
    zsum = z_ref[0] + z_ref[1] + hsin_ref[...]
    xn = jnp.maximum(zsum * dis_ref[...] + b_ref[...], 0.0)
    h = lax.dot_general(xn, w_ref[...], (((1,), (0,)), ((), ())),
                        preferred_element_type=jnp.float32,
                        precision=lax.Precision.HIGHEST)
    hs_ref[...] = h * dis_ref[...]


def _tc_pool_body(z_ref, hsin_ref, dis_ref, b_ref, batch_ref, out_ref):
    zsum = z_ref[0] + z_ref[1] + hsin_ref[...]
    h3 = (zsum * dis_ref[...] + b_ref[...])[:N, :]
    gi = lax.broadcasted_iota(jnp.int32, (G, N), 0)
    P = jnp.where(gi == batch_ref[...], 1.0, 0.0)
    sums = lax.dot_general(P, h3, (((1,), (0,)), ((), ())),
                           preferred_element_type=jnp.float32,
                           precision=lax.Precision.HIGHEST)
    cnt = jnp.sum(P, axis=1, keepdims=True)
    out_ref[...] = sums / jnp.maximum(cnt, 1.0)


_tc_first = pl.pallas_call(
    _tc_first_body, out_shape=jax.ShapeDtypeStruct((NP, D), jnp.float32))
_tc_mid = pl.pallas_call(
    _tc_mid_body, out_shape=jax.ShapeDtypeStruct((NP, D), jnp.float32))
_tc_pool = pl.pallas_call(
    _tc_pool_body, out_shape=jax.ShapeDtypeStruct((G, D), jnp.float32))


@jax.jit
def kernel(x, edge_index, batch, W1, b1, W2, b2, W3, b3):
    src = edge_index[0].astype(jnp.int32)
    dst = edge_index[1].astype(jnp.int32)
    pad = jnp.full((EPAD - E,), N, jnp.int32)
    srcp = jnp.concatenate([src, pad]).reshape(NT, CPT, CH)
    dstp = jnp.concatenate([dst, pad]).reshape(NT, CPT, CH)

    x0 = jnp.zeros((NP, D), jnp.float32).at[:N].set(x)
    zer = jnp.zeros((NP, D), jnp.float32)
    batch2d = batch.astype(jnp.int32).reshape(1, N)

    dis = _sc_deg(dstp)
    dis2d = dis.reshape(NP, 1)

    hs = _tc_first(x0, W1, dis2d)
    z = _sc_agg(srcp, dstp, hs, zer)
    hs = _tc_mid(z, hs, dis2d, W2, b1.reshape(1, D))
    z = _sc_agg(srcp, dstp, hs, zer)
    hs = _tc_mid(z, hs, dis2d, W3, b2.reshape(1, D))
    z = _sc_agg(srcp, dstp, hs, zer)
    return _tc_pool(z, hs, dis2d, b3.reshape(1, D), batch2d)


# speedup vs baseline: 1.0004x; 1.0004x over previous
import functools

import jax
import jax.numpy as jnp
from jax import lax
from jax.experimental import pallas as pl
from jax.experimental.pallas import tpu as pltpu
from jax.experimental.pallas import tpu_sc as plsc

N = 10000
NP = 10240
E = 320000
D = 128
G = 64
NC, NS, L = 2, 16, 16
NT = NC * NS
CH = 128
CPT = 80
EPAD = NT * CPT * CH
NB = 2
NGR = CPT // NB
RPT = NP // NS

_mesh = plsc.VectorSubcoreMesh(core_axis_name="c", subcore_axis_name="s")


def _newton_rsqrt(d):
    i = plsc.bitcast(d, jnp.int32)
    i = jnp.int32(0x5F3759DF) - lax.shift_right_logical(i, 1)
    y = plsc.bitcast(i, jnp.float32)
    for _ in range(4):
        y = y * (1.5 - 0.5 * d * y * y)
    return y


@functools.partial(
    pl.kernel,
    out_type=jax.ShapeDtypeStruct((NP,), jnp.float32),
    mesh=_mesh,
    compiler_params=pltpu.CompilerParams(needs_layout_passes=False),
    scratch_types=[
        pltpu.VMEM((NP,), jnp.float32),
        pltpu.VMEM((CPT, CH), jnp.int32),
        pltpu.VMEM((RPT,), jnp.float32),
        pltpu.VMEM((RPT,), jnp.float32),
        pltpu.VMEM_SHARED((NS, NP), jnp.float32),
    ],
)
def _sc_deg(dstp_ref, dis_ref, hist, dstbuf, acc, tmp, hsh):
    c = lax.axis_index("c")
    s = lax.axis_index("s")
    ones = jnp.ones((L,), jnp.float32)

    @pl.when(c == 0)
    def _():
        def zhist(i, _):
            hist[pl.ds(i * L, L)] = jnp.zeros((L,), jnp.float32)
            return _
        lax.fori_loop(0, NP // L, zhist, 0)

        for t in range(2):
            w = s + t * NS
            pltpu.sync_copy(dstp_ref.at[w], dstbuf)

            def brow(r, _):
                for jj in range(CH // L):
                    idx = dstbuf[r, pl.ds(jj * L, L)]
                    plsc.addupdate_scatter(hist, [idx], ones)
                return _
            lax.fori_loop(0, CPT, brow, 0)

        pltpu.sync_copy(hist, hsh.at[s])
        plsc.subcore_barrier()

        base = s * RPT
        def zacc(i, _):
            acc[pl.ds(i * L, L)] = jnp.zeros((L,), jnp.float32)
            return _
        lax.fori_loop(0, RPT // L, zacc, 0)
        for k in range(NS):
            pltpu.sync_copy(hsh.at[k, pl.ds(base, RPT)], tmp)

            def radd(i, _):
                sl = pl.ds(i * L, L)
                acc[sl] = acc[sl] + tmp[sl]
                return _
            lax.fori_loop(0, RPT // L, radd, 0)

        def wdis(i, _):
            sl = pl.ds(i * L, L)
            deg = acc[sl] + 1.0
            y = _newton_rsqrt(deg)
            ids = base + i * L + lax.iota(jnp.int32, L)
            tmp[sl] = jnp.where(ids < N, y, 0.0)
            return _
        lax.fori_loop(0, RPT // L, wdis, 0)
        pltpu.sync_copy(tmp, dis_ref.at[pl.ds(base, RPT)])


@functools.partial(
    pl.kernel,
    out_type=jax.ShapeDtypeStruct((NC, NP, D), jnp.float32),
    mesh=_mesh,
    scratch_types=(
        [pltpu.VMEM((CH, D), jnp.float32) for _ in range(NB)]
        + [pltpu.VMEM((CH,), jnp.int32) for _ in range(NB)]
        + [pltpu.VMEM((CH,), jnp.int32) for _ in range(NB)]
        + [pltpu.SemaphoreType.DMA for _ in range(NB)]
        + [pltpu.VMEM_SHARED((NP, D), jnp.float32)]
    ),
)
def _sc_agg(srcp_ref, dstp_ref, hs_ref, zer_ref, z_ref, *scr):
    rows = scr[0:NB]
    isrc = scr[NB:2 * NB]
    idst = scr[2 * NB:3 * NB]
    gsem = scr[3 * NB:4 * NB]
    z_sh = scr[4 * NB]

    c = lax.axis_index("c")
    s = lax.axis_index("s")
    w = c * NS + s
    base = s * RPT

    with jax.named_scope("agg_seed"):
        for k in range(RPT // CH):
            pltpu.sync_copy(zer_ref.at[pl.ds(base + k * CH, CH)], rows[0])
            pltpu.sync_copy(rows[0], z_sh.at[pl.ds(base + k * CH, CH)])

        plsc.subcore_barrier()

    with jax.named_scope("agg_prime"):
        for b in range(NB):
            pltpu.sync_copy(srcp_ref.at[w, b], isrc[b])
            pltpu.sync_copy(dstp_ref.at[w, b], idst[b])
            pltpu.async_copy(hs_ref.at[isrc[b]], rows[b], gsem[b])

    def group(g, carry):
        for b in range(NB):
            pltpu.make_async_copy(hs_ref.at[isrc[b]], rows[b], gsem[b]).wait()
            pltpu.sync_copy(rows[b], z_sh.at[idst[b]], add=True)

            @pl.when(g < NGR - 1)
            def _(b=b):
                jn = (g + 1) * NB + b
                pltpu.sync_copy(srcp_ref.at[w, jn], isrc[b])
                pltpu.sync_copy(dstp_ref.at[w, jn], idst[b])
                pltpu.async_copy(hs_ref.at[isrc[b]], rows[b], gsem[b])
        return carry

    with jax.named_scope("agg_main"):
        lax.fori_loop(0, NGR, group, 0)
        plsc.subcore_barrier()

    with jax.named_scope("agg_out"):
        for k in range(RPT // CH):
            sl = pl.ds(base + k * CH, CH)
            pltpu.sync_copy(z_sh.at[sl], z_ref.at[c, sl])


def _tc_first_body(x_ref, w_ref, dis_ref, hs_ref):
    h = lax.dot_general(x_ref[...], w_ref[...], (((1,), (0,)), ((), ())),
                        preferred_element_type=jnp.float32,
                        precision=lax.Precision.HIGHEST)
    hs_ref[...] = h * dis_ref[...]


def _tc_mid_body(z_ref, hsin_ref, dis_ref, w_ref, b_ref, hs_ref):
    zsum = z_ref[0] + z_ref[1] + hsin_ref[...]
    xn = jnp.maximum(zsum * dis_ref[...] + b_ref[...], 0.0)
    h = lax.dot_general(xn, w_ref[...], (((1,), (0,)), ((), ())),
                        preferred_element_type=jnp.float32,
                        precision=lax.Precision.HIGHEST)
    hs_ref[...] = h * dis_ref[...]


def _tc_pool_body(z_ref, hsin_ref, dis_ref, b_ref, batch_ref, out_ref):
    zsum = z_ref[0] + z_ref[1] + hsin_ref[...]
    h3 = (zsum * dis_ref[...] + b_ref[...])[:N, :]
    gi = lax.broadcasted_iota(jnp.int32, (G, N), 0)
    P = jnp.where(gi == batch_ref[...], 1.0, 0.0)
    sums = lax.dot_general(P, h3, (((1,), (0,)), ((), ())),
                           preferred_element_type=jnp.float32,
                           precision=lax.Precision.HIGHEST)
    cnt = jnp.sum(P, axis=1, keepdims=True)
    out_ref[...] = sums / jnp.maximum(cnt, 1.0)


_tc_first = pl.pallas_call(
    _tc_first_body, out_shape=jax.ShapeDtypeStruct((NP, D), jnp.float32))
_tc_mid = pl.pallas_call(
    _tc_mid_body, out_shape=jax.ShapeDtypeStruct((NP, D), jnp.float32))
_tc_pool = pl.pallas_call(
    _tc_pool_body, out_shape=jax.ShapeDtypeStruct((G, D), jnp.float32))


@jax.jit
def kernel(x, edge_index, batch, W1, b1, W2, b2, W3, b3):
    src = edge_index[0].astype(jnp.int32)
    dst = edge_index[1].astype(jnp.int32)
    pad = jnp.full((EPAD - E,), N, jnp.int32)
    srcp = jnp.concatenate([src, pad]).reshape(NT, CPT, CH)
    dstp = jnp.concatenate([dst, pad]).reshape(NT, CPT, CH)

    x0 = jnp.zeros((NP, D), jnp.float32).at[:N].set(x)
    zer = jnp.zeros((NP, D), jnp.float32)
    batch2d = batch.astype(jnp.int32).reshape(1, N)

    dis = _sc_deg(dstp)
    dis2d = dis.reshape(NP, 1)

    hs = _tc_first(x0, W1, dis2d)
    z = _sc_agg(srcp, dstp, hs, zer)
    hs = _tc_mid(z, hs, dis2d, W2, b1.reshape(1, D))
    z = _sc_agg(srcp, dstp, hs, zer)
    hs = _tc_mid(z, hs, dis2d, W3, b2.reshape(1, D))
    z = _sc_agg(srcp, dstp, hs, zer)
    return _tc_pool(z, hs, dis2d, b3.reshape(1, D), batch2d)


# speedup vs baseline: 1.0834x; 1.0830x over previous
import functools

import jax
import jax.numpy as jnp
from jax import lax
from jax.experimental import pallas as pl
from jax.experimental.pallas import tpu as pltpu
from jax.experimental.pallas import tpu_sc as plsc

N = 10000
NP = 10240
E = 320000
D = 128
G = 64
NC, NS, L = 2, 16, 16
NT = NC * NS
CH = 128
CPT = 80
EPAD = NT * CPT * CH
NB = 2
NGR = CPT // NB
RPT = NP // NS

_mesh = plsc.VectorSubcoreMesh(core_axis_name="c", subcore_axis_name="s")


def _newton_rsqrt(d):
    i = plsc.bitcast(d, jnp.int32)
    i = jnp.int32(0x5F3759DF) - lax.shift_right_logical(i, 1)
    y = plsc.bitcast(i, jnp.float32)
    for _ in range(4):
        y = y * (1.5 - 0.5 * d * y * y)
    return y


@functools.partial(
    pl.kernel,
    out_type=jax.ShapeDtypeStruct((NP,), jnp.float32),
    mesh=_mesh,
    compiler_params=pltpu.CompilerParams(needs_layout_passes=False),
    scratch_types=[
        pltpu.VMEM((NP,), jnp.float32),
        pltpu.VMEM((CPT, CH), jnp.int32),
        pltpu.VMEM((RPT,), jnp.float32),
        pltpu.VMEM((RPT,), jnp.float32),
        pltpu.VMEM_SHARED((NS, NP), jnp.float32),
    ],
)
def _sc_deg(dstp_ref, dis_ref, hist, dstbuf, acc, tmp, hsh):
    c = lax.axis_index("c")
    s = lax.axis_index("s")
    ones = jnp.ones((L,), jnp.float32)

    @pl.when(c == 0)
    def _():
        def zhist(i, _):
            hist[pl.ds(i * L, L)] = jnp.zeros((L,), jnp.float32)
            return _
        lax.fori_loop(0, NP // L, zhist, 0)

        for t in range(2):
            w = s + t * NS
            pltpu.sync_copy(dstp_ref.at[w], dstbuf)

            def brow(r, _):
                for jj in range(CH // L):
                    idx = dstbuf[r, pl.ds(jj * L, L)]
                    plsc.addupdate_scatter(hist, [idx], ones)
                return _
            lax.fori_loop(0, CPT, brow, 0)

        pltpu.sync_copy(hist, hsh.at[s])
        plsc.subcore_barrier()

        base = s * RPT
        def zacc(i, _):
            acc[pl.ds(i * L, L)] = jnp.zeros((L,), jnp.float32)
            return _
        lax.fori_loop(0, RPT // L, zacc, 0)
        for k in range(NS):
            pltpu.sync_copy(hsh.at[k, pl.ds(base, RPT)], tmp)

            def radd(i, _):
                sl = pl.ds(i * L, L)
                acc[sl] = acc[sl] + tmp[sl]
                return _
            lax.fori_loop(0, RPT // L, radd, 0)

        def wdis(i, _):
            sl = pl.ds(i * L, L)
            deg = acc[sl] + 1.0
            y = _newton_rsqrt(deg)
            ids = base + i * L + lax.iota(jnp.int32, L)
            tmp[sl] = jnp.where(ids < N, y, 0.0)
            return _
        lax.fori_loop(0, RPT // L, wdis, 0)
        pltpu.sync_copy(tmp, dis_ref.at[pl.ds(base, RPT)])


SBC = 8
NSB = CPT // SBC


@functools.partial(
    pl.kernel,
    out_type=jax.ShapeDtypeStruct((NC, NP, D), jnp.float32),
    mesh=_mesh,
    scratch_types=(
        [pltpu.VMEM((CH, D), jnp.float32) for _ in range(NB)]
        + [pltpu.VMEM((SBC, CH), jnp.int32) for _ in range(2)]
        + [pltpu.VMEM((SBC, CH), jnp.int32) for _ in range(2)]
        + [pltpu.SemaphoreType.DMA for _ in range(NB)]
        + [pltpu.SemaphoreType.DMA for _ in range(2)]
        + [pltpu.SemaphoreType.DMA]
        + [pltpu.VMEM_SHARED((NP, D), jnp.float32)]
    ),
)
def _sc_agg(srcp_ref, dstp_ref, hs_ref, z_ref, *scr):
    rows = scr[0:NB]
    isrcS = scr[NB:NB + 2]
    idstS = scr[NB + 2:NB + 4]
    gsem = scr[NB + 4:2 * NB + 4]
    isem = scr[2 * NB + 4:2 * NB + 6]
    osem = scr[2 * NB + 6]
    z_sh = scr[2 * NB + 7]

    c = lax.axis_index("c")
    s = lax.axis_index("s")
    w = c * NS + s
    base = s * RPT

    with jax.named_scope("agg_seed"):
        def zrow(i, carry):
            for jj in range(D // L):
                rows[0][i, pl.ds(jj * L, L)] = jnp.zeros((L,), jnp.float32)
            return carry
        lax.fori_loop(0, CH, zrow, 0)
        for k in range(RPT // CH):
            pltpu.sync_copy(rows[0], z_sh.at[pl.ds(base + k * CH, CH)])
        plsc.subcore_barrier()

    with jax.named_scope("agg_prime"):
        for sb in range(2):
            pltpu.async_copy(srcp_ref.at[w, pl.ds(sb * SBC, SBC)],
                             isrcS[sb], isem[sb])
            pltpu.async_copy(dstp_ref.at[w, pl.ds(sb * SBC, SBC)],
                             idstS[sb], isem[sb])
        pltpu.make_async_copy(srcp_ref.at[w, pl.ds(0, SBC)],
                              isrcS[0], isem[0]).wait()
        pltpu.make_async_copy(dstp_ref.at[w, pl.ds(0, SBC)],
                              idstS[0], isem[0]).wait()
        for b in range(NB):
            pltpu.async_copy(hs_ref.at[isrcS[0].at[b]], rows[b], gsem[b])

    def sb2body(p, carry):
        for q in range(2):
            sb = 2 * p + q

            @pl.when(sb > 0)
            def _(q=q, sb=sb):
                pltpu.make_async_copy(srcp_ref.at[w, pl.ds(sb * SBC, SBC)],
                                      isrcS[q], isem[q]).wait()
                pltpu.make_async_copy(dstp_ref.at[w, pl.ds(sb * SBC, SBC)],
                                      idstS[q], isem[q]).wait()

            @pl.when((sb >= 1) & (sb <= NSB - 2))
            def _(q=q, sb=sb):
                nq = 1 - q
                pltpu.async_copy(srcp_ref.at[w, pl.ds((sb + 1) * SBC, SBC)],
                                 isrcS[nq], isem[nq])
                pltpu.async_copy(dstp_ref.at[w, pl.ds((sb + 1) * SBC, SBC)],
                                 idstS[nq], isem[nq])

            for j2 in range(SBC):
                b = j2 % NB
                pltpu.make_async_copy(hs_ref.at[isrcS[q].at[j2]],
                                      rows[b], gsem[b]).wait()
                pltpu.sync_copy(rows[b], z_sh.at[idstS[q].at[j2]], add=True)
                if j2 < SBC - NB:
                    pltpu.async_copy(hs_ref.at[isrcS[q].at[j2 + NB]],
                                     rows[b], gsem[b])
                else:
                    @pl.when(sb < NSB - 1)
                    def _(q=q, j2=j2, b=b):
                        pltpu.async_copy(
                            hs_ref.at[isrcS[1 - q].at[j2 + NB - SBC]],
                            rows[b], gsem[b])
        return carry

    with jax.named_scope("agg_main"):
        lax.fori_loop(0, NSB // 2, sb2body, 0)
        plsc.subcore_barrier()

    with jax.named_scope("agg_out"):
        for k in range(RPT // CH):
            sl = pl.ds(base + k * CH, CH)
            pltpu.async_copy(z_sh.at[sl], z_ref.at[c, sl], osem)
        for k in range(RPT // CH):
            sl = pl.ds(base + k * CH, CH)
            pltpu.make_async_copy(z_sh.at[sl], z_ref.at[c, sl], osem).wait()


def _tc_first_body(x_ref, w_ref, dis_ref, hs_ref):
    h = lax.dot_general(x_ref[...], w_ref[...], (((1,), (0,)), ((), ())),
                        preferred_element_type=jnp.float32,
                        precision=lax.Precision.HIGHEST)
    hs_ref[...] = h * dis_ref[...]


def _tc_mid_body(z_ref, hsin_ref, dis_ref, w_ref, b_ref, hs_ref):
    zsum = z_ref[0] + z_ref[1] + hsin_ref[...]
    xn = jnp.maximum(zsum * dis_ref[...] + b_ref[...], 0.0)
    h = lax.dot_general(xn, w_ref[...], (((1,), (0,)), ((), ())),
                        preferred_element_type=jnp.float32,
                        precision=lax.Precision.HIGHEST)
    hs_ref[...] = h * dis_ref[...]


def _tc_pool_body(z_ref, hsin_ref, dis_ref, b_ref, batch_ref, out_ref):
    zsum = z_ref[0] + z_ref[1] + hsin_ref[...]
    h3 = (zsum * dis_ref[...] + b_ref[...])[:N, :]
    gi = lax.broadcasted_iota(jnp.int32, (G, N), 0)
    P = jnp.where(gi == batch_ref[...], 1.0, 0.0)
    sums = lax.dot_general(P, h3, (((1,), (0,)), ((), ())),
                           preferred_element_type=jnp.float32,
                           precision=lax.Precision.HIGHEST)
    cnt = jnp.sum(P, axis=1, keepdims=True)
    out_ref[...] = sums / jnp.maximum(cnt, 1.0)


_tc_first = pl.pallas_call(
    _tc_first_body, out_shape=jax.ShapeDtypeStruct((NP, D), jnp.float32))
_tc_mid = pl.pallas_call(
    _tc_mid_body, out_shape=jax.ShapeDtypeStruct((NP, D), jnp.float32))
_tc_pool = pl.pallas_call(
    _tc_pool_body, out_shape=jax.ShapeDtypeStruct((G, D), jnp.float32))


@jax.jit
def kernel(x, edge_index, batch, W1, b1, W2, b2, W3, b3):
    src = edge_index[0].astype(jnp.int32)
    dst = edge_index[1].astype(jnp.int32)
    pad = jnp.full((EPAD - E,), N, jnp.int32)
    srcp = jnp.concatenate([src, pad]).reshape(NT, CPT, CH)
    dstp = jnp.concatenate([dst, pad]).reshape(NT, CPT, CH)

    x0 = jnp.zeros((NP, D), jnp.float32).at[:N].set(x)
    batch2d = batch.astype(jnp.int32).reshape(1, N)

    dis = _sc_deg(dstp)
    dis2d = dis.reshape(NP, 1)

    hs = _tc_first(x0, W1, dis2d)
    z = _sc_agg(srcp, dstp, hs)
    hs = _tc_mid(z, hs, dis2d, W2, b1.reshape(1, D))
    z = _sc_agg(srcp, dstp, hs)
    hs = _tc_mid(z, hs, dis2d, W3, b2.reshape(1, D))
    z = _sc_agg(srcp, dstp, hs)
    return _tc_pool(z, hs, dis2d, b3.reshape(1, D), batch2d)


# speedup vs baseline: 3.5893x; 3.3129x over previous
import functools

import jax
import jax.numpy as jnp
from jax import lax
from jax.experimental import pallas as pl
from jax.experimental.pallas import tpu as pltpu
from jax.experimental.pallas import tpu_sc as plsc

N = 10000
NP = 10240
E = 320000
D = 128
G = 64
NC, NS, L = 2, 16, 16
NT = NC * NS
CH = 128
CPT = 80
EPAD = NT * CPT * CH
NB = 2
NGR = CPT // NB
RPT = NP // NS

_mesh = plsc.VectorSubcoreMesh(core_axis_name="c", subcore_axis_name="s")


def _newton_rsqrt(d):
    i = plsc.bitcast(d, jnp.int32)
    i = jnp.int32(0x5F3759DF) - lax.shift_right_logical(i, 1)
    y = plsc.bitcast(i, jnp.float32)
    for _ in range(4):
        y = y * (1.5 - 0.5 * d * y * y)
    return y


@functools.partial(
    pl.kernel,
    out_type=jax.ShapeDtypeStruct((NP,), jnp.float32),
    mesh=_mesh,
    compiler_params=pltpu.CompilerParams(needs_layout_passes=False),
    scratch_types=[
        pltpu.VMEM((NP,), jnp.float32),
        pltpu.VMEM((CPT, CH), jnp.int32),
        pltpu.VMEM((RPT,), jnp.float32),
        pltpu.VMEM((RPT,), jnp.float32),
        pltpu.VMEM_SHARED((NS, NP), jnp.float32),
    ],
)
def _sc_deg(dstp_ref, dis_ref, hist, dstbuf, acc, tmp, hsh):
    c = lax.axis_index("c")
    s = lax.axis_index("s")
    ones = jnp.ones((L,), jnp.float32)

    @pl.when(c == 0)
    def _():
        def zhist(i, _):
            hist[pl.ds(i * L, L)] = jnp.zeros((L,), jnp.float32)
            return _
        lax.fori_loop(0, NP // L, zhist, 0)

        for t in range(2):
            w = s + t * NS
            pltpu.sync_copy(dstp_ref.at[w], dstbuf)

            def brow(r, _):
                for jj in range(CH // L):
                    idx = dstbuf[r, pl.ds(jj * L, L)]
                    plsc.addupdate_scatter(hist, [idx], ones)
                return _
            lax.fori_loop(0, CPT, brow, 0)

        pltpu.sync_copy(hist, hsh.at[s])
        plsc.subcore_barrier()

        base = s * RPT
        def zacc(i, _):
            acc[pl.ds(i * L, L)] = jnp.zeros((L,), jnp.float32)
            return _
        lax.fori_loop(0, RPT // L, zacc, 0)
        for k in range(NS):
            pltpu.sync_copy(hsh.at[k, pl.ds(base, RPT)], tmp)

            def radd(i, _):
                sl = pl.ds(i * L, L)
                acc[sl] = acc[sl] + tmp[sl]
                return _
            lax.fori_loop(0, RPT // L, radd, 0)

        def wdis(i, _):
            sl = pl.ds(i * L, L)
            deg = acc[sl] + 1.0
            y = _newton_rsqrt(deg)
            ids = base + i * L + lax.iota(jnp.int32, L)
            tmp[sl] = jnp.where(ids < N, y, 0.0)
            return _
        lax.fori_loop(0, RPT // L, wdis, 0)
        pltpu.sync_copy(tmp, dis_ref.at[pl.ds(base, RPT)])


SBC = 8
NSB = CPT // SBC


@functools.partial(
    pl.kernel,
    out_type=jax.ShapeDtypeStruct((NC, NP, D), jnp.float32),
    mesh=_mesh,
    scratch_types=(
        [pltpu.VMEM((CH, D), jnp.float32) for _ in range(NB)]
        + [pltpu.VMEM((SBC, CH), jnp.int32) for _ in range(2)]
        + [pltpu.VMEM((SBC, CH), jnp.int32) for _ in range(2)]
        + [pltpu.SemaphoreType.DMA for _ in range(NB)]
        + [pltpu.SemaphoreType.DMA for _ in range(2)]
        + [pltpu.SemaphoreType.DMA]
        + [pltpu.VMEM_SHARED((NP, D), jnp.float32)]
    ),
)
def _sc_agg(srcp_ref, dstp_ref, hs_ref, z_ref, *scr):
    rows = scr[0:NB]
    isrcS = scr[NB:NB + 2]
    idstS = scr[NB + 2:NB + 4]
    gsem = scr[NB + 4:2 * NB + 4]
    isem = scr[2 * NB + 4:2 * NB + 6]
    osem = scr[2 * NB + 6]
    z_sh = scr[2 * NB + 7]

    c = lax.axis_index("c")
    s = lax.axis_index("s")
    w = c * NS + s
    base = s * RPT

    with jax.named_scope("agg_seed"):
        def zrow(i, carry):
            for jj in range(D // L):
                rows[0][i, pl.ds(jj * L, L)] = jnp.zeros((L,), jnp.float32)
            return carry
        lax.fori_loop(0, CH, zrow, 0)
        for k in range(RPT // CH):
            pltpu.sync_copy(rows[0], z_sh.at[pl.ds(base + k * CH, CH)])
        plsc.subcore_barrier()

    with jax.named_scope("agg_prime"):
        for sb in range(2):
            pltpu.async_copy(srcp_ref.at[w, pl.ds(sb * SBC, SBC)],
                             isrcS[sb], isem[sb])
            pltpu.async_copy(dstp_ref.at[w, pl.ds(sb * SBC, SBC)],
                             idstS[sb], isem[sb])
        pltpu.make_async_copy(srcp_ref.at[w, pl.ds(0, SBC)],
                              isrcS[0], isem[0]).wait()
        pltpu.make_async_copy(dstp_ref.at[w, pl.ds(0, SBC)],
                              idstS[0], isem[0]).wait()
        for b in range(NB):
            pltpu.async_copy(hs_ref.at[isrcS[0].at[b]], rows[b], gsem[b])

    def sb2body(p, carry):
        for q in range(2):
            sb = 2 * p + q

            @pl.when(sb > 0)
            def _(q=q, sb=sb):
                pltpu.make_async_copy(srcp_ref.at[w, pl.ds(sb * SBC, SBC)],
                                      isrcS[q], isem[q]).wait()
                pltpu.make_async_copy(dstp_ref.at[w, pl.ds(sb * SBC, SBC)],
                                      idstS[q], isem[q]).wait()

            @pl.when((sb >= 1) & (sb <= NSB - 2))
            def _(q=q, sb=sb):
                nq = 1 - q
                pltpu.async_copy(srcp_ref.at[w, pl.ds((sb + 1) * SBC, SBC)],
                                 isrcS[nq], isem[nq])
                pltpu.async_copy(dstp_ref.at[w, pl.ds((sb + 1) * SBC, SBC)],
                                 idstS[nq], isem[nq])

            for j2 in range(SBC):
                b = j2 % NB
                pltpu.make_async_copy(hs_ref.at[isrcS[q].at[j2]],
                                      rows[b], gsem[b]).wait()
                pltpu.sync_copy(rows[b], z_sh.at[idstS[q].at[j2]], add=True)
                if j2 < SBC - NB:
                    pltpu.async_copy(hs_ref.at[isrcS[q].at[j2 + NB]],
                                     rows[b], gsem[b])
                else:
                    @pl.when(sb < NSB - 1)
                    def _(q=q, j2=j2, b=b):
                        pltpu.async_copy(
                            hs_ref.at[isrcS[1 - q].at[j2 + NB - SBC]],
                            rows[b], gsem[b])
        return carry

    with jax.named_scope("agg_main"):
        lax.fori_loop(0, NSB // 2, sb2body, 0)
        plsc.subcore_barrier()

    with jax.named_scope("agg_out"):
        for k in range(RPT // CH):
            sl = pl.ds(base + k * CH, CH)
            pltpu.async_copy(z_sh.at[sl], z_ref.at[c, sl], osem)
        for k in range(RPT // CH):
            sl = pl.ds(base + k * CH, CH)
            pltpu.make_async_copy(z_sh.at[sl], z_ref.at[c, sl], osem).wait()


def _tc_first_body(x_ref, w_ref, dis_ref, hs_ref):
    h = lax.dot_general(x_ref[...], w_ref[...], (((1,), (0,)), ((), ())),
                        preferred_element_type=jnp.float32,
                        precision=lax.Precision.HIGHEST)
    hs_ref[...] = h * dis_ref[...]


def _tc_mid_body(z_ref, hsin_ref, dis_ref, w_ref, b_ref, hs_ref):
    zsum = z_ref[0] + z_ref[1] + hsin_ref[...]
    xn = jnp.maximum(zsum * dis_ref[...] + b_ref[...], 0.0)
    h = lax.dot_general(xn, w_ref[...], (((1,), (0,)), ((), ())),
                        preferred_element_type=jnp.float32,
                        precision=lax.Precision.HIGHEST)
    hs_ref[...] = h * dis_ref[...]


def _tc_pool_body(z_ref, hsin_ref, dis_ref, b_ref, batch_ref, out_ref):
    zsum = z_ref[0] + z_ref[1] + hsin_ref[...]
    h3 = (zsum * dis_ref[...] + b_ref[...])[:N, :]
    gi = lax.broadcasted_iota(jnp.int32, (G, N), 0)
    P = jnp.where(gi == batch_ref[...], 1.0, 0.0)
    sums = lax.dot_general(P, h3, (((1,), (0,)), ((), ())),
                           preferred_element_type=jnp.float32,
                           precision=lax.Precision.HIGHEST)
    cnt = jnp.sum(P, axis=1, keepdims=True)
    out_ref[...] = sums / jnp.maximum(cnt, 1.0)


_tc_first = pl.pallas_call(
    _tc_first_body, out_shape=jax.ShapeDtypeStruct((NP, D), jnp.float32))
_tc_mid = pl.pallas_call(
    _tc_mid_body, out_shape=jax.ShapeDtypeStruct((NP, D), jnp.float32))
_tc_pool = pl.pallas_call(
    _tc_pool_body, out_shape=jax.ShapeDtypeStruct((G, D), jnp.float32))


@jax.jit
def kernel(x, edge_index, batch, W1, b1, W2, b2, W3, b3):
    src = edge_index[0].astype(jnp.int32)
    dst = edge_index[1].astype(jnp.int32)
    pad = N + jnp.arange(EPAD - E, dtype=jnp.int32) % (NP - N)
    srcp = jnp.concatenate([src, pad]).reshape(NT, CPT, CH)
    dstp = jnp.concatenate([dst, pad]).reshape(NT, CPT, CH)

    x0 = jnp.zeros((NP, D), jnp.float32).at[:N].set(x)
    batch2d = batch.astype(jnp.int32).reshape(1, N)

    dis = _sc_deg(dstp)
    dis2d = dis.reshape(NP, 1)

    hs = _tc_first(x0, W1, dis2d)
    z = _sc_agg(srcp, dstp, hs)
    hs = _tc_mid(z, hs, dis2d, W2, b1.reshape(1, D))
    z = _sc_agg(srcp, dstp, hs)
    hs = _tc_mid(z, hs, dis2d, W3, b2.reshape(1, D))
    z = _sc_agg(srcp, dstp, hs)
    return _tc_pool(z, hs, dis2d, b3.reshape(1, D), batch2d)
